# baseline (speedup 1.0000x reference)
import functools

import jax
import jax.numpy as jnp
from jax import lax
from jax.experimental import pallas as pl
from jax.experimental.pallas import tpu as pltpu
from jax.experimental.pallas import tpu_sc as plsc

BATCH = 16384
D = 64
NV = 1000000
NP = 500224
NC = 2
NS = 16
L = 16
NW = NC * NS
BPW = BATCH // NW
CH = 256
NR = BPW // CH
IC = 128

DT_LANES = 1024
DT_GRID = (NV + DT_LANES - 1) // DT_LANES


def _detile_body(in_ref, out_ref):
    x = in_ref[...]
    h = DT_LANES // 2
    eye = jnp.eye(D, dtype=jnp.float32)
    dn = (((0,), (0,)), ((), ()))

    def t(block):
        return lax.dot_general(block, eye, dn,
                               preferred_element_type=jnp.float32)

    out_ref[...] = jnp.concatenate([t(x[:, :h]), t(x[:, h:])], axis=1)


def _detile(tab_t):
    return pl.pallas_call(
        _detile_body,
        grid=(DT_GRID,),
        in_specs=[pl.BlockSpec((D, DT_LANES), lambda i: (0, i))],
        out_specs=pl.BlockSpec((DT_LANES // 2, 2 * D), lambda i: (i, 0)),
        out_shape=jax.ShapeDtypeStruct((NP, 2 * D), jnp.float32),
    )(tab_t)


def _sc_body(uidx_hbm, pidx_hbm, nidx_hbm, utab_hbm, itab_hbm,
             pos_out, neg_out, sq_out,
             uidx_v, pidx_v, nidx_v, ugat, pgat, ngat, ubuf, pbuf, nbuf,
             posb, negb, sqb, sem):
    wid = lax.axis_index("s") * NC + lax.axis_index("c")
    base = wid * BPW

    lane = lax.iota(jnp.int32, L)
    zero = jnp.zeros((L,), jnp.float32)

    def round_body(r, _):
        src = pl.ds(base + r * CH, CH)
        pltpu.sync_copy(uidx_hbm.at[src], uidx_v)
        pltpu.sync_copy(pidx_hbm.at[src], pidx_v)
        pltpu.sync_copy(nidx_hbm.at[src], nidx_v)

        def _row_of(e):
            return lax.shift_left(lax.shift_right_logical(e, 10), 9) + \
                jnp.bitwise_and(e, 511)

        def make_rows(j, _):
            sl = pl.ds(j * L, L)
            ugat[sl] = _row_of(uidx_v[sl])
            pgat[sl] = _row_of(pidx_v[sl])
            ngat[sl] = _row_of(nidx_v[sl])
            return 0

        lax.fori_loop(0, CH // L, make_rows, 0)

        copies = []
        for k in range(CH // IC):
            dst = pl.ds(k * IC, IC)
            isl = pl.ds(k * IC, IC)
            copies.append(pltpu.async_copy(utab_hbm.at[ugat.at[isl]], ubuf.at[dst], sem))
            copies.append(pltpu.async_copy(itab_hbm.at[pgat.at[isl]], pbuf.at[dst], sem))
            copies.append(pltpu.async_copy(itab_hbm.at[ngat.at[isl]], nbuf.at[dst], sem))
        for c in copies:
            c.wait()

        def group(g, _):
            sl16 = pl.ds(g * L, L)
            rows = g * L + lane
            offu = lax.shift_left(
                jnp.bitwise_and(lax.shift_right_logical(uidx_v[sl16], 9), 1), 6)
            offp = lax.shift_left(
                jnp.bitwise_and(lax.shift_right_logical(pidx_v[sl16], 9), 1), 6)
            offn = lax.shift_left(
                jnp.bitwise_and(lax.shift_right_logical(nidx_v[sl16], 9), 1), 6)
            ap = zero
            an = zero
            asq = zero
            for c in range(D):
                cc = jnp.full((L,), c, jnp.int32)
                u = plsc.load_gather(ubuf, [rows, offu + cc])
                p = plsc.load_gather(pbuf, [rows, offp + cc])
                q = plsc.load_gather(nbuf, [rows, offn + cc])
                ap = ap + u * p
                an = an + u * q
                asq = asq + (u * u + (p * p + q * q))
            osl = pl.ds(r * CH + g * L, L)
            posb[osl] = ap
            negb[osl] = an
            sqb[osl] = asq
            return 0

        lax.fori_loop(0, CH // L, group, 0)
        return 0

    lax.fori_loop(0, NR, round_body, 0)

    out_sl = pl.ds(base, BPW)
    pltpu.sync_copy(posb, pos_out.at[out_sl])
    pltpu.sync_copy(negb, neg_out.at[out_sl])
    pltpu.sync_copy(sqb, sq_out.at[out_sl])


_sc_dots = functools.partial(
    pl.kernel,
    out_type=[
        jax.ShapeDtypeStruct((BATCH,), jnp.float32),
        jax.ShapeDtypeStruct((BATCH,), jnp.float32),
        jax.ShapeDtypeStruct((BATCH,), jnp.float32),
    ],
    mesh=plsc.VectorSubcoreMesh(
        core_axis_name="c", subcore_axis_name="s", num_cores=NC, num_subcores=NS
    ),
    compiler_params=pltpu.CompilerParams(
        needs_layout_passes=False, use_tc_tiling_on_sc=False
    ),
    scratch_types=[
        pltpu.VMEM((CH,), jnp.int32),
        pltpu.VMEM((CH,), jnp.int32),
        pltpu.VMEM((CH,), jnp.int32),
        pltpu.VMEM((CH,), jnp.int32),
        pltpu.VMEM((CH,), jnp.int32),
        pltpu.VMEM((CH,), jnp.int32),
        pltpu.VMEM((CH, 2 * D), jnp.float32),
        pltpu.VMEM((CH, 2 * D), jnp.float32),
        pltpu.VMEM((CH, 2 * D), jnp.float32),
        pltpu.VMEM((BPW,), jnp.float32),
        pltpu.VMEM((BPW,), jnp.float32),
        pltpu.VMEM((BPW,), jnp.float32),
        pltpu.SemaphoreType.DMA,
    ],
)(_sc_body)


def _tc_loss_body(pos_ref, neg_ref, sq_ref, out_ref):
    pos = pos_ref[...]
    neg = neg_ref[...]
    sp = 1.0 / (1.0 + jnp.exp(-pos))
    sn = 1.0 / (1.0 + jnp.exp(-neg))
    z = sp - sn
    cf = jnp.mean(jnp.log(1.0 + jnp.exp(-z)))
    reg = 0.5 * jnp.mean(sq_ref[...])
    out_ref[0, 0] = cf + 1e-4 * reg


def kernel(user_indices, pos_item_indices, neg_item_indices, user_table, item_table):
    uidx = user_indices.astype(jnp.int32)
    pidx = pos_item_indices.astype(jnp.int32)
    nidx = neg_item_indices.astype(jnp.int32)

    gu = _detile(user_table.T)
    gi = _detile(item_table.T)

    pos_d, neg_d, sq_d = _sc_dots(uidx, pidx, nidx, gu, gi)

    loss = pl.pallas_call(
        _tc_loss_body,
        out_shape=jax.ShapeDtypeStruct((1, 1), jnp.float32),
        out_specs=pl.BlockSpec(memory_space=pltpu.SMEM),
    )(
        pos_d.reshape(128, 128),
        neg_d.reshape(128, 128),
        sq_d.reshape(128, 128),
    )
    return loss[0, 0]


# speedup vs baseline: 2.0832x; 2.0832x over previous
import functools

import jax
import jax.numpy as jnp
from jax import lax
from jax.experimental import pallas as pl
from jax.experimental.pallas import tpu as pltpu
from jax.experimental.pallas import tpu_sc as plsc

BATCH = 16384
D = 64
NV = 1000000
NP = 501760
NC = 2
NS = 16
L = 16
NW = NC * NS
BPW = BATCH // NW
CH = 256
NR = BPW // CH
IC = 128

DT_LANES = 4096
DT_GRID = (NV + DT_LANES - 1) // DT_LANES


def _detile_body(in_ref, out_ref):
    x = in_ref[...]
    h = DT_LANES // 2
    eye = jnp.eye(D, dtype=jnp.float32)
    dn = (((0,), (0,)), ((), ()))

    def t(block):
        return lax.dot_general(block, eye, dn,
                               preferred_element_type=jnp.float32)

    out_ref[...] = jnp.concatenate([t(x[:, :h]), t(x[:, h:])], axis=1)


def _detile(tab_t):
    return pl.pallas_call(
        _detile_body,
        grid=(DT_GRID,),
        in_specs=[pl.BlockSpec((D, DT_LANES), lambda i: (0, i))],
        out_specs=pl.BlockSpec((DT_LANES // 2, 2 * D), lambda i: (i, 0)),
        out_shape=jax.ShapeDtypeStruct((NP, 2 * D), jnp.float32),
    )(tab_t)


def _sc_body(uidx_hbm, pidx_hbm, nidx_hbm, utab_hbm, itab_hbm,
             pos_out, neg_out, sq_out,
             uidx_v, pidx_v, nidx_v, ugat, pgat, ngat, ubuf, pbuf, nbuf,
             posb, negb, sqb, sem):
    wid = lax.axis_index("s") * NC + lax.axis_index("c")
    base = wid * BPW

    lane = lax.iota(jnp.int32, L)
    zero = jnp.zeros((L,), jnp.float32)

    def round_body(r, _):
        src = pl.ds(base + r * CH, CH)
        pltpu.sync_copy(uidx_hbm.at[src], uidx_v)
        pltpu.sync_copy(pidx_hbm.at[src], pidx_v)
        pltpu.sync_copy(nidx_hbm.at[src], nidx_v)

        def _row_of(e):
            return lax.shift_left(lax.shift_right_logical(e, 12), 11) + \
                jnp.bitwise_and(e, 2047)

        def make_rows(j, _):
            sl = pl.ds(j * L, L)
            ugat[sl] = _row_of(uidx_v[sl])
            pgat[sl] = _row_of(pidx_v[sl])
            ngat[sl] = _row_of(nidx_v[sl])
            return 0

        lax.fori_loop(0, CH // L, make_rows, 0)

        copies = []
        for k in range(CH // IC):
            dst = pl.ds(k * IC, IC)
            isl = pl.ds(k * IC, IC)
            copies.append(pltpu.async_copy(utab_hbm.at[ugat.at[isl]], ubuf.at[dst], sem))
            copies.append(pltpu.async_copy(itab_hbm.at[pgat.at[isl]], pbuf.at[dst], sem))
            copies.append(pltpu.async_copy(itab_hbm.at[ngat.at[isl]], nbuf.at[dst], sem))
        for c in copies:
            c.wait()

        def group(g, _):
            sl16 = pl.ds(g * L, L)
            rows = g * L + lane
            offu = lax.shift_left(
                jnp.bitwise_and(lax.shift_right_logical(uidx_v[sl16], 11), 1), 6)
            offp = lax.shift_left(
                jnp.bitwise_and(lax.shift_right_logical(pidx_v[sl16], 11), 1), 6)
            offn = lax.shift_left(
                jnp.bitwise_and(lax.shift_right_logical(nidx_v[sl16], 11), 1), 6)
            ap = zero
            an = zero
            asq = zero
            for c in range(D):
                cc = jnp.full((L,), c, jnp.int32)
                u = plsc.load_gather(ubuf, [rows, offu + cc])
                p = plsc.load_gather(pbuf, [rows, offp + cc])
                q = plsc.load_gather(nbuf, [rows, offn + cc])
                ap = ap + u * p
                an = an + u * q
                asq = asq + (u * u + (p * p + q * q))
            osl = pl.ds(r * CH + g * L, L)
            posb[osl] = ap
            negb[osl] = an
            sqb[osl] = asq
            return 0

        lax.fori_loop(0, CH // L, group, 0)
        return 0

    lax.fori_loop(0, NR, round_body, 0)

    out_sl = pl.ds(base, BPW)
    pltpu.sync_copy(posb, pos_out.at[out_sl])
    pltpu.sync_copy(negb, neg_out.at[out_sl])
    pltpu.sync_copy(sqb, sq_out.at[out_sl])


_sc_dots = functools.partial(
    pl.kernel,
    out_type=[
        jax.ShapeDtypeStruct((BATCH,), jnp.float32),
        jax.ShapeDtypeStruct((BATCH,), jnp.float32),
        jax.ShapeDtypeStruct((BATCH,), jnp.float32),
    ],
    mesh=plsc.VectorSubcoreMesh(
        core_axis_name="c", subcore_axis_name="s", num_cores=NC, num_subcores=NS
    ),
    compiler_params=pltpu.CompilerParams(
        needs_layout_passes=False, use_tc_tiling_on_sc=False
    ),
    scratch_types=[
        pltpu.VMEM((CH,), jnp.int32),
        pltpu.VMEM((CH,), jnp.int32),
        pltpu.VMEM((CH,), jnp.int32),
        pltpu.VMEM((CH,), jnp.int32),
        pltpu.VMEM((CH,), jnp.int32),
        pltpu.VMEM((CH,), jnp.int32),
        pltpu.VMEM((CH, 2 * D), jnp.float32),
        pltpu.VMEM((CH, 2 * D), jnp.float32),
        pltpu.VMEM((CH, 2 * D), jnp.float32),
        pltpu.VMEM((BPW,), jnp.float32),
        pltpu.VMEM((BPW,), jnp.float32),
        pltpu.VMEM((BPW,), jnp.float32),
        pltpu.SemaphoreType.DMA,
    ],
)(_sc_body)


def _tc_loss_body(pos_ref, neg_ref, sq_ref, out_ref):
    pos = pos_ref[...]
    neg = neg_ref[...]
    sp = 1.0 / (1.0 + jnp.exp(-pos))
    sn = 1.0 / (1.0 + jnp.exp(-neg))
    z = sp - sn
    cf = jnp.mean(jnp.log(1.0 + jnp.exp(-z)))
    reg = 0.5 * jnp.mean(sq_ref[...])
    out_ref[0, 0] = cf + 1e-4 * reg


def kernel(user_indices, pos_item_indices, neg_item_indices, user_table, item_table):
    uidx = user_indices.astype(jnp.int32)
    pidx = pos_item_indices.astype(jnp.int32)
    nidx = neg_item_indices.astype(jnp.int32)

    gu = _detile(user_table.T)
    gi = _detile(item_table.T)

    pos_d, neg_d, sq_d = _sc_dots(uidx, pidx, nidx, gu, gi)

    loss = pl.pallas_call(
        _tc_loss_body,
        out_shape=jax.ShapeDtypeStruct((1, 1), jnp.float32),
        out_specs=pl.BlockSpec(memory_space=pltpu.SMEM),
    )(
        pos_d.reshape(128, 128),
        neg_d.reshape(128, 128),
        sq_d.reshape(128, 128),
    )
    return loss[0, 0]


# speedup vs baseline: 2.8874x; 1.3860x over previous
import functools

import jax
import jax.numpy as jnp
from jax import lax
from jax.experimental import pallas as pl
from jax.experimental.pallas import tpu as pltpu
from jax.experimental.pallas import tpu_sc as plsc

BATCH = 16384
D = 64
NV = 1000000
NP = 507904
NC = 2
NS = 16
L = 16
NW = NC * NS
BPW = BATCH // NW
CH = 256
NR = BPW // CH
IC = 128

DT_LANES = 16384
DT_GRID = (NV + DT_LANES - 1) // DT_LANES


def _detile_body(in_ref, out_ref):
    x = in_ref[...]
    h = DT_LANES // 2
    eye = jnp.eye(D, dtype=jnp.float32)
    dn = (((0,), (0,)), ((), ()))

    def t(block):
        return lax.dot_general(block, eye, dn,
                               preferred_element_type=jnp.float32)

    out_ref[...] = jnp.concatenate([t(x[:, :h]), t(x[:, h:])], axis=1)


def _detile(tab_t):
    return pl.pallas_call(
        _detile_body,
        grid=(DT_GRID,),
        in_specs=[pl.BlockSpec((D, DT_LANES), lambda i: (0, i))],
        out_specs=pl.BlockSpec((DT_LANES // 2, 2 * D), lambda i: (i, 0)),
        out_shape=jax.ShapeDtypeStruct((NP, 2 * D), jnp.float32),
    )(tab_t)


def _sc_body(uidx_hbm, pidx_hbm, nidx_hbm, utab_hbm, itab_hbm,
             pos_out, neg_out, sq_out,
             uidx_v, pidx_v, nidx_v, ugat, pgat, ngat, ubuf, pbuf, nbuf,
             posb, negb, sqb, sem):
    wid = lax.axis_index("s") * NC + lax.axis_index("c")
    base = wid * BPW

    lane = lax.iota(jnp.int32, L)
    zero = jnp.zeros((L,), jnp.float32)

    def round_body(r, _):
        src = pl.ds(base + r * CH, CH)
        pltpu.sync_copy(uidx_hbm.at[src], uidx_v)
        pltpu.sync_copy(pidx_hbm.at[src], pidx_v)
        pltpu.sync_copy(nidx_hbm.at[src], nidx_v)

        def _row_of(e):
            return lax.shift_left(lax.shift_right_logical(e, 14), 13) + \
                jnp.bitwise_and(e, 8191)

        def make_rows(j, _):
            sl = pl.ds(j * L, L)
            ugat[sl] = _row_of(uidx_v[sl])
            pgat[sl] = _row_of(pidx_v[sl])
            ngat[sl] = _row_of(nidx_v[sl])
            return 0

        lax.fori_loop(0, CH // L, make_rows, 0)

        copies = []
        for k in range(CH // IC):
            dst = pl.ds(k * IC, IC)
            isl = pl.ds(k * IC, IC)
            copies.append(pltpu.async_copy(utab_hbm.at[ugat.at[isl]], ubuf.at[dst], sem))
            copies.append(pltpu.async_copy(itab_hbm.at[pgat.at[isl]], pbuf.at[dst], sem))
            copies.append(pltpu.async_copy(itab_hbm.at[ngat.at[isl]], nbuf.at[dst], sem))
        for c in copies:
            c.wait()

        def group(g, _):
            sl16 = pl.ds(g * L, L)
            rows = g * L + lane
            offu = lax.shift_left(
                jnp.bitwise_and(lax.shift_right_logical(uidx_v[sl16], 13), 1), 6)
            offp = lax.shift_left(
                jnp.bitwise_and(lax.shift_right_logical(pidx_v[sl16], 13), 1), 6)
            offn = lax.shift_left(
                jnp.bitwise_and(lax.shift_right_logical(nidx_v[sl16], 13), 1), 6)
            ap = zero
            an = zero
            asq = zero
            for c in range(D):
                cc = jnp.full((L,), c, jnp.int32)
                u = plsc.load_gather(ubuf, [rows, offu + cc])
                p = plsc.load_gather(pbuf, [rows, offp + cc])
                q = plsc.load_gather(nbuf, [rows, offn + cc])
                ap = ap + u * p
                an = an + u * q
                asq = asq + (u * u + (p * p + q * q))
            osl = pl.ds(r * CH + g * L, L)
            posb[osl] = ap
            negb[osl] = an
            sqb[osl] = asq
            return 0

        lax.fori_loop(0, CH // L, group, 0)
        return 0

    lax.fori_loop(0, NR, round_body, 0)

    out_sl = pl.ds(base, BPW)
    pltpu.sync_copy(posb, pos_out.at[out_sl])
    pltpu.sync_copy(negb, neg_out.at[out_sl])
    pltpu.sync_copy(sqb, sq_out.at[out_sl])


_sc_dots = functools.partial(
    pl.kernel,
    out_type=[
        jax.ShapeDtypeStruct((BATCH,), jnp.float32),
        jax.ShapeDtypeStruct((BATCH,), jnp.float32),
        jax.ShapeDtypeStruct((BATCH,), jnp.float32),
    ],
    mesh=plsc.VectorSubcoreMesh(
        core_axis_name="c", subcore_axis_name="s", num_cores=NC, num_subcores=NS
    ),
    compiler_params=pltpu.CompilerParams(
        needs_layout_passes=False, use_tc_tiling_on_sc=False
    ),
    scratch_types=[
        pltpu.VMEM((CH,), jnp.int32),
        pltpu.VMEM((CH,), jnp.int32),
        pltpu.VMEM((CH,), jnp.int32),
        pltpu.VMEM((CH,), jnp.int32),
        pltpu.VMEM((CH,), jnp.int32),
        pltpu.VMEM((CH,), jnp.int32),
        pltpu.VMEM((CH, 2 * D), jnp.float32),
        pltpu.VMEM((CH, 2 * D), jnp.float32),
        pltpu.VMEM((CH, 2 * D), jnp.float32),
        pltpu.VMEM((BPW,), jnp.float32),
        pltpu.VMEM((BPW,), jnp.float32),
        pltpu.VMEM((BPW,), jnp.float32),
        pltpu.SemaphoreType.DMA,
    ],
)(_sc_body)


def _tc_loss_body(pos_ref, neg_ref, sq_ref, out_ref):
    pos = pos_ref[...]
    neg = neg_ref[...]
    sp = 1.0 / (1.0 + jnp.exp(-pos))
    sn = 1.0 / (1.0 + jnp.exp(-neg))
    z = sp - sn
    cf = jnp.mean(jnp.log(1.0 + jnp.exp(-z)))
    reg = 0.5 * jnp.mean(sq_ref[...])
    out_ref[0, 0] = cf + 1e-4 * reg


def kernel(user_indices, pos_item_indices, neg_item_indices, user_table, item_table):
    uidx = user_indices.astype(jnp.int32)
    pidx = pos_item_indices.astype(jnp.int32)
    nidx = neg_item_indices.astype(jnp.int32)

    gu = _detile(user_table.T)
    gi = _detile(item_table.T)

    pos_d, neg_d, sq_d = _sc_dots(uidx, pidx, nidx, gu, gi)

    loss = pl.pallas_call(
        _tc_loss_body,
        out_shape=jax.ShapeDtypeStruct((1, 1), jnp.float32),
        out_specs=pl.BlockSpec(memory_space=pltpu.SMEM),
    )(
        pos_d.reshape(128, 128),
        neg_d.reshape(128, 128),
        sq_d.reshape(128, 128),
    )
    return loss[0, 0]


# speedup vs baseline: 3.0551x; 1.0581x over previous
import functools

import jax
import jax.numpy as jnp
from jax import lax
from jax.experimental import pallas as pl
from jax.experimental.pallas import tpu as pltpu
from jax.experimental.pallas import tpu_sc as plsc

BATCH = 16384
D = 64
NV = 1000000
NP = 507904
NC = 2
NS = 16
L = 16
NW = NC * NS
BPW = BATCH // NW
CH = 256
NR = BPW // CH
IC = 128

DT_LANES = 32768
DT_GRID = (NV + DT_LANES - 1) // DT_LANES


def _detile_body(in_ref, out_ref):
    x = in_ref[...]
    h = DT_LANES // 2
    eye = jnp.eye(D, dtype=jnp.float32)
    dn = (((0,), (0,)), ((), ()))

    def t(block):
        return lax.dot_general(block, eye, dn,
                               preferred_element_type=jnp.float32)

    out_ref[...] = jnp.concatenate([t(x[:, :h]), t(x[:, h:])], axis=1)


def _detile(tab_t):
    return pl.pallas_call(
        _detile_body,
        grid=(DT_GRID,),
        in_specs=[pl.BlockSpec((D, DT_LANES), lambda i: (0, i))],
        out_specs=pl.BlockSpec((DT_LANES // 2, 2 * D), lambda i: (i, 0)),
        out_shape=jax.ShapeDtypeStruct((NP, 2 * D), jnp.float32),
    )(tab_t)


def _sc_body(uidx_hbm, pidx_hbm, nidx_hbm, utab_hbm, itab_hbm,
             pos_out, neg_out, sq_out,
             uidx_v, pidx_v, nidx_v, ugat, pgat, ngat, ubuf, pbuf, nbuf,
             posb, negb, sqb, sem):
    wid = lax.axis_index("s") * NC + lax.axis_index("c")
    base = wid * BPW

    lane = lax.iota(jnp.int32, L)
    zero = jnp.zeros((L,), jnp.float32)

    def round_body(r, _):
        src = pl.ds(base + r * CH, CH)
        pltpu.sync_copy(uidx_hbm.at[src], uidx_v)
        pltpu.sync_copy(pidx_hbm.at[src], pidx_v)
        pltpu.sync_copy(nidx_hbm.at[src], nidx_v)

        def _row_of(e):
            return lax.shift_left(lax.shift_right_logical(e, 15), 14) + \
                jnp.bitwise_and(e, 16383)

        def make_rows(j, _):
            sl = pl.ds(j * L, L)
            ugat[sl] = _row_of(uidx_v[sl])
            pgat[sl] = _row_of(pidx_v[sl])
            ngat[sl] = _row_of(nidx_v[sl])
            return 0

        lax.fori_loop(0, CH // L, make_rows, 0)

        copies = []
        for k in range(CH // IC):
            dst = pl.ds(k * IC, IC)
            isl = pl.ds(k * IC, IC)
            copies.append(pltpu.async_copy(utab_hbm.at[ugat.at[isl]], ubuf.at[dst], sem))
            copies.append(pltpu.async_copy(itab_hbm.at[pgat.at[isl]], pbuf.at[dst], sem))
            copies.append(pltpu.async_copy(itab_hbm.at[ngat.at[isl]], nbuf.at[dst], sem))
        for c in copies:
            c.wait()

        def group(g, _):
            sl16 = pl.ds(g * L, L)
            rows = g * L + lane
            offu = lax.shift_left(
                jnp.bitwise_and(lax.shift_right_logical(uidx_v[sl16], 14), 1), 6)
            offp = lax.shift_left(
                jnp.bitwise_and(lax.shift_right_logical(pidx_v[sl16], 14), 1), 6)
            offn = lax.shift_left(
                jnp.bitwise_and(lax.shift_right_logical(nidx_v[sl16], 14), 1), 6)
            ap = zero
            an = zero
            asq = zero
            for c in range(D):
                cc = jnp.full((L,), c, jnp.int32)
                u = plsc.load_gather(ubuf, [rows, offu + cc])
                p = plsc.load_gather(pbuf, [rows, offp + cc])
                q = plsc.load_gather(nbuf, [rows, offn + cc])
                ap = ap + u * p
                an = an + u * q
                asq = asq + (u * u + (p * p + q * q))
            osl = pl.ds(r * CH + g * L, L)
            posb[osl] = ap
            negb[osl] = an
            sqb[osl] = asq
            return 0

        lax.fori_loop(0, CH // L, group, 0)
        return 0

    lax.fori_loop(0, NR, round_body, 0)

    out_sl = pl.ds(base, BPW)
    pltpu.sync_copy(posb, pos_out.at[out_sl])
    pltpu.sync_copy(negb, neg_out.at[out_sl])
    pltpu.sync_copy(sqb, sq_out.at[out_sl])


_sc_dots = functools.partial(
    pl.kernel,
    out_type=[
        jax.ShapeDtypeStruct((BATCH,), jnp.float32),
        jax.ShapeDtypeStruct((BATCH,), jnp.float32),
        jax.ShapeDtypeStruct((BATCH,), jnp.float32),
    ],
    mesh=plsc.VectorSubcoreMesh(
        core_axis_name="c", subcore_axis_name="s", num_cores=NC, num_subcores=NS
    ),
    compiler_params=pltpu.CompilerParams(
        needs_layout_passes=False, use_tc_tiling_on_sc=False
    ),
    scratch_types=[
        pltpu.VMEM((CH,), jnp.int32),
        pltpu.VMEM((CH,), jnp.int32),
        pltpu.VMEM((CH,), jnp.int32),
        pltpu.VMEM((CH,), jnp.int32),
        pltpu.VMEM((CH,), jnp.int32),
        pltpu.VMEM((CH,), jnp.int32),
        pltpu.VMEM((CH, 2 * D), jnp.float32),
        pltpu.VMEM((CH, 2 * D), jnp.float32),
        pltpu.VMEM((CH, 2 * D), jnp.float32),
        pltpu.VMEM((BPW,), jnp.float32),
        pltpu.VMEM((BPW,), jnp.float32),
        pltpu.VMEM((BPW,), jnp.float32),
        pltpu.SemaphoreType.DMA,
    ],
)(_sc_body)


def _tc_loss_body(pos_ref, neg_ref, sq_ref, out_ref):
    pos = pos_ref[...]
    neg = neg_ref[...]
    sp = 1.0 / (1.0 + jnp.exp(-pos))
    sn = 1.0 / (1.0 + jnp.exp(-neg))
    z = sp - sn
    cf = jnp.mean(jnp.log(1.0 + jnp.exp(-z)))
    reg = 0.5 * jnp.mean(sq_ref[...])
    out_ref[0, 0] = cf + 1e-4 * reg


def kernel(user_indices, pos_item_indices, neg_item_indices, user_table, item_table):
    uidx = user_indices.astype(jnp.int32)
    pidx = pos_item_indices.astype(jnp.int32)
    nidx = neg_item_indices.astype(jnp.int32)

    gu = _detile(user_table.T)
    gi = _detile(item_table.T)

    pos_d, neg_d, sq_d = _sc_dots(uidx, pidx, nidx, gu, gi)

    loss = pl.pallas_call(
        _tc_loss_body,
        out_shape=jax.ShapeDtypeStruct((1, 1), jnp.float32),
        out_specs=pl.BlockSpec(memory_space=pltpu.SMEM),
    )(
        pos_d.reshape(128, 128),
        neg_d.reshape(128, 128),
        sq_d.reshape(128, 128),
    )
    return loss[0, 0]


# speedup vs baseline: 3.0642x; 1.0030x over previous
import functools

import jax
import jax.numpy as jnp
from jax import lax
from jax.experimental import pallas as pl
from jax.experimental.pallas import tpu as pltpu
from jax.experimental.pallas import tpu_sc as plsc

BATCH = 16384
D = 64
NV = 1000000
NP = 507904
NC = 2
NS = 16
L = 16
NW = NC * NS
BPW = BATCH // NW
CH = 256
NR = BPW // CH
IC = 128

DT_LANES = 32768
DT_GRID = (NV + DT_LANES - 1) // DT_LANES


def _detile_body(in_ref, out_ref):
    x = in_ref[...]
    h = DT_LANES // 2
    out_ref[...] = jnp.concatenate(
        [jnp.transpose(x[:, :h]), jnp.transpose(x[:, h:])], axis=1
    )


def _detile(tab_t):
    return pl.pallas_call(
        _detile_body,
        grid=(DT_GRID,),
        in_specs=[pl.BlockSpec((D, DT_LANES), lambda i: (0, i))],
        out_specs=pl.BlockSpec((DT_LANES // 2, 2 * D), lambda i: (i, 0)),
        out_shape=jax.ShapeDtypeStruct((NP, 2 * D), jnp.float32),
    )(tab_t)


def _sc_body(uidx_hbm, pidx_hbm, nidx_hbm, utab_hbm, itab_hbm,
             pos_out, neg_out, sq_out,
             uidx_v, pidx_v, nidx_v, ugat, pgat, ngat, ubuf, pbuf, nbuf,
             posb, negb, sqb, sem):
    wid = lax.axis_index("s") * NC + lax.axis_index("c")
    base = wid * BPW

    lane = lax.iota(jnp.int32, L)
    zero = jnp.zeros((L,), jnp.float32)

    def round_body(r, _):
        src = pl.ds(base + r * CH, CH)
        pltpu.sync_copy(uidx_hbm.at[src], uidx_v)
        pltpu.sync_copy(pidx_hbm.at[src], pidx_v)
        pltpu.sync_copy(nidx_hbm.at[src], nidx_v)

        def _row_of(e):
            return lax.shift_left(lax.shift_right_logical(e, 15), 14) + \
                jnp.bitwise_and(e, 16383)

        def make_rows(j, _):
            sl = pl.ds(j * L, L)
            ugat[sl] = _row_of(uidx_v[sl])
            pgat[sl] = _row_of(pidx_v[sl])
            ngat[sl] = _row_of(nidx_v[sl])
            return 0

        lax.fori_loop(0, CH // L, make_rows, 0)

        copies = []
        for k in range(CH // IC):
            dst = pl.ds(k * IC, IC)
            isl = pl.ds(k * IC, IC)
            copies.append(pltpu.async_copy(utab_hbm.at[ugat.at[isl]], ubuf.at[dst], sem))
            copies.append(pltpu.async_copy(itab_hbm.at[pgat.at[isl]], pbuf.at[dst], sem))
            copies.append(pltpu.async_copy(itab_hbm.at[ngat.at[isl]], nbuf.at[dst], sem))
        for c in copies:
            c.wait()

        def group(g, _):
            sl16 = pl.ds(g * L, L)
            rows = g * L + lane
            offu = lax.shift_left(
                jnp.bitwise_and(lax.shift_right_logical(uidx_v[sl16], 14), 1), 6)
            offp = lax.shift_left(
                jnp.bitwise_and(lax.shift_right_logical(pidx_v[sl16], 14), 1), 6)
            offn = lax.shift_left(
                jnp.bitwise_and(lax.shift_right_logical(nidx_v[sl16], 14), 1), 6)
            ap = zero
            an = zero
            asq = zero
            for c in range(D):
                cc = jnp.full((L,), c, jnp.int32)
                u = plsc.load_gather(ubuf, [rows, offu + cc])
                p = plsc.load_gather(pbuf, [rows, offp + cc])
                q = plsc.load_gather(nbuf, [rows, offn + cc])
                ap = ap + u * p
                an = an + u * q
                asq = asq + (u * u + (p * p + q * q))
            osl = pl.ds(r * CH + g * L, L)
            posb[osl] = ap
            negb[osl] = an
            sqb[osl] = asq
            return 0

        lax.fori_loop(0, CH // L, group, 0)
        return 0

    lax.fori_loop(0, NR, round_body, 0)

    out_sl = pl.ds(base, BPW)
    pltpu.sync_copy(posb, pos_out.at[out_sl])
    pltpu.sync_copy(negb, neg_out.at[out_sl])
    pltpu.sync_copy(sqb, sq_out.at[out_sl])


_sc_dots = functools.partial(
    pl.kernel,
    out_type=[
        jax.ShapeDtypeStruct((BATCH,), jnp.float32),
        jax.ShapeDtypeStruct((BATCH,), jnp.float32),
        jax.ShapeDtypeStruct((BATCH,), jnp.float32),
    ],
    mesh=plsc.VectorSubcoreMesh(
        core_axis_name="c", subcore_axis_name="s", num_cores=NC, num_subcores=NS
    ),
    compiler_params=pltpu.CompilerParams(
        needs_layout_passes=False, use_tc_tiling_on_sc=False
    ),
    scratch_types=[
        pltpu.VMEM((CH,), jnp.int32),
        pltpu.VMEM((CH,), jnp.int32),
        pltpu.VMEM((CH,), jnp.int32),
        pltpu.VMEM((CH,), jnp.int32),
        pltpu.VMEM((CH,), jnp.int32),
        pltpu.VMEM((CH,), jnp.int32),
        pltpu.VMEM((CH, 2 * D), jnp.float32),
        pltpu.VMEM((CH, 2 * D), jnp.float32),
        pltpu.VMEM((CH, 2 * D), jnp.float32),
        pltpu.VMEM((BPW,), jnp.float32),
        pltpu.VMEM((BPW,), jnp.float32),
        pltpu.VMEM((BPW,), jnp.float32),
        pltpu.SemaphoreType.DMA,
    ],
)(_sc_body)


def _tc_loss_body(pos_ref, neg_ref, sq_ref, out_ref):
    pos = pos_ref[...]
    neg = neg_ref[...]
    sp = 1.0 / (1.0 + jnp.exp(-pos))
    sn = 1.0 / (1.0 + jnp.exp(-neg))
    z = sp - sn
    cf = jnp.mean(jnp.log(1.0 + jnp.exp(-z)))
    reg = 0.5 * jnp.mean(sq_ref[...])
    out_ref[0, 0] = cf + 1e-4 * reg


def kernel(user_indices, pos_item_indices, neg_item_indices, user_table, item_table):
    uidx = user_indices.astype(jnp.int32)
    pidx = pos_item_indices.astype(jnp.int32)
    nidx = neg_item_indices.astype(jnp.int32)

    gu = _detile(user_table.T)
    gi = _detile(item_table.T)

    pos_d, neg_d, sq_d = _sc_dots(uidx, pidx, nidx, gu, gi)

    loss = pl.pallas_call(
        _tc_loss_body,
        out_shape=jax.ShapeDtypeStruct((1, 1), jnp.float32),
        out_specs=pl.BlockSpec(memory_space=pltpu.SMEM),
    )(
        pos_d.reshape(128, 128),
        neg_d.reshape(128, 128),
        sq_d.reshape(128, 128),
    )
    return loss[0, 0]
